# baseline (speedup 1.0000x reference)
import functools

import jax
import jax.numpy as jnp
from jax import lax
from jax.experimental import pallas as pl
from jax.experimental.pallas import tpu as pltpu
from jax.experimental.pallas import tpu_sc as plsc

N = 10000
E = 320000
D = 128
NC = 2
NS = 16
NPAD = 10240
RPT = NPAD // NS
CK = 128
TOTCH = 2560
EPAD = TOTCH * CK
N0 = 80
N1 = 80
NMAX = max(N0, N1)
NCH = TOTCH // NS
HK = CK // 2

import numpy as _np
_PERM = _np.concatenate([
    _np.stack([32 * b + _np.arange(16), 32 * b + 16 + _np.arange(16)],
              axis=1).reshape(32)
    for b in range(4)
])

_mesh = plsc.VectorSubcoreMesh(
    core_axis_name="c", subcore_axis_name="s", num_cores=NC, num_subcores=NS)


def _sc_deg_body(dst_hbm, ew_hbm, dinv_hbm, shdeg, dstv, ewv, buf):
    c = lax.axis_index("c")
    s = lax.axis_index("s")

    @pl.when(c == 0)
    def _():
        rs = s * RPT
        for k in range(RPT // 16):
            buf[pl.ds(k * 16, 16)] = jnp.zeros((16,), jnp.float32)
        pltpu.sync_copy(buf, shdeg.at[pl.ds(rs, RPT)])
        plsc.subcore_barrier()

        pltpu.sync_copy(dst_hbm.at[pl.ds(s * NCH, NCH)], dstv)
        pltpu.sync_copy(ew_hbm.at[pl.ds(s * NCH, NCH)], ewv)

        def chunk(j, carry):
            pltpu.sync_copy(ewv.at[j], shdeg.at[dstv.at[j]], add=True)
            return carry

        lax.fori_loop(0, NCH, chunk, 0)
        plsc.subcore_barrier()

        pltpu.sync_copy(shdeg.at[pl.ds(rs, RPT)], buf)
        for k in range(RPT // 16):
            v = buf[pl.ds(k * 16, 16)] + 1.0
            i = lax.bitcast_convert_type(v, jnp.int32)
            g = lax.bitcast_convert_type(
                jnp.int32(0x5F3759DF) - (i >> 1), jnp.float32)
            for _ in range(3):
                g = g * (1.5 - 0.5 * v * g * g)
            buf[pl.ds(k * 16, 16)] = g
        pltpu.sync_copy(buf, dinv_hbm.at[pl.ds(rs, RPT)])


_sc_deg = functools.partial(
    pl.kernel,
    out_type=jax.ShapeDtypeStruct((NPAD,), jnp.float32),
    mesh=_mesh,
    scratch_types=[
        pltpu.VMEM_SHARED((NPAD,), jnp.float32),
        pltpu.VMEM((NCH, CK), jnp.int32),
        pltpu.VMEM((NCH, CK), jnp.float32),
        pltpu.VMEM((RPT,), jnp.float32),
    ],
)(_sc_deg_body)


def _sc_layer_body(y_hbm, src_hbm, dst_hbm, ew_hbm, out_hbm,
                   shacc, srcv, dstv, ewv, rows,
                   semg, semd, semw):
    c = lax.axis_index("c")
    s = lax.axis_index("s")
    rs = s * RPT

    def zrow(r, carry):
        for k in range(D // 16):
            rows[0, r, pl.ds(k * 16, 16)] = jnp.zeros((16,), jnp.float32)
        return carry

    lax.fori_loop(0, CK, zrow, 0)
    for t in range(RPT // CK):
        pltpu.sync_copy(rows.at[0], shacc.at[pl.ds(rs + t * CK, CK)])

    n_c = jnp.where(c == 0, N0, N1)
    qbase = c * (NS * N0) + s * n_c

    if N0 > 0:
        @pl.when(c == 0)
        def _():
            pltpu.sync_copy(
                src_hbm.at[pl.ds(qbase, N0)], srcv.at[pl.ds(0, N0)])

    if N1 > 0:
        @pl.when(c == 1)
        def _():
            pltpu.sync_copy(
                src_hbm.at[pl.ds(qbase, N1)], srcv.at[pl.ds(0, N1)])

    plsc.subcore_barrier()

    def issue(j, b):
        pltpu.async_copy(y_hbm.at[srcv.at[j]], rows.at[b], semg)
        pltpu.async_copy(dst_hbm.at[qbase + j], dstv.at[b], semd)
        pltpu.async_copy(ew_hbm.at[qbase + j], ewv.at[b], semw)

    def wait_in(b):
        pltpu.make_async_copy(y_hbm.at[srcv.at[0]], rows.at[b], semg).wait()
        pltpu.make_async_copy(dst_hbm.at[0], dstv.at[b], semd).wait()
        pltpu.make_async_copy(ew_hbm.at[0], ewv.at[b], semw).wait()

    def scale(b):
        def edge16(eb, c2):
            wv = ewv[b, pl.ds(eb * 16, 16)]
            for l in range(16):
                w = wv[l]
                e = eb * 16 + l
                for k in range(D // 16):
                    rows[b, e, pl.ds(k * 16, 16)] = (
                        rows[b, e, pl.ds(k * 16, 16)] * w)
            return c2

        lax.fori_loop(0, CK // 16, edge16, 0)

    npairs = n_c // 2

    @pl.when(npairs > 0)
    def _():
        issue(0, 0)

    def pair(p, carry):
        j0 = 2 * p
        wait_in(0)
        issue(j0 + 1, 1)
        scale(0)
        pltpu.sync_copy(rows.at[0], shacc.at[dstv.at[0]], add=True)
        wait_in(1)

        @pl.when(p < npairs - 1)
        def _():
            issue(j0 + 2, 0)

        scale(1)
        pltpu.sync_copy(rows.at[1], shacc.at[dstv.at[1]], add=True)
        return carry

    lax.fori_loop(0, npairs, pair, 0)
    plsc.subcore_barrier()
    pltpu.sync_copy(shacc.at[pl.ds(rs, RPT)], out_hbm.at[c, pl.ds(rs, RPT)])


_sc_layer = functools.partial(
    pl.kernel,
    out_type=jax.ShapeDtypeStruct((NC, NPAD, D), jnp.float32),
    mesh=_mesh,
    scratch_types=[
        pltpu.VMEM_SHARED((NPAD, D), jnp.float32),
        pltpu.VMEM((NMAX, CK), jnp.int32),
        pltpu.VMEM((2, CK), jnp.int32),
        pltpu.VMEM((2, CK), jnp.float32),
        pltpu.VMEM((2, CK, D), jnp.float32),
        pltpu.SemaphoreType.DMA,
        pltpu.SemaphoreType.DMA,
        pltpu.SemaphoreType.DMA,
    ],
)(_sc_layer_body)


GRID_R = 8
BR = NPAD // GRID_R


def _tc_a_body(x_ref, w_ref, d_ref, y_ref):
    xw = jnp.dot(x_ref[...], w_ref[...], preferred_element_type=jnp.float32)
    y_ref[...] = xw * d_ref[...]


def _tc_a(xp, W1, dinv2d):
    return pl.pallas_call(
        _tc_a_body,
        grid=(GRID_R,),
        in_specs=[
            pl.BlockSpec((BR, D), lambda i: (i, 0)),
            pl.BlockSpec((D, D), lambda i: (0, 0)),
            pl.BlockSpec((BR, 1), lambda i: (i, 0)),
        ],
        out_specs=pl.BlockSpec((BR, D), lambda i: (i, 0)),
        out_shape=jax.ShapeDtypeStruct((NPAD, D), jnp.float32),
    )(xp, W1, dinv2d)


def _tc_b_body(a_ref, y_ref, d_ref, b_ref, w_ref, o_ref):
    d = d_ref[...]
    h = jnp.maximum(
        (a_ref[0] + a_ref[1] + y_ref[...]) * d + b_ref[...], 0.0)
    o_ref[...] = jnp.dot(
        h, w_ref[...], preferred_element_type=jnp.float32) * d


def _tc_b(acc1, y1, dinv2d, b1, W2):
    return pl.pallas_call(
        _tc_b_body,
        grid=(GRID_R,),
        in_specs=[
            pl.BlockSpec((NC, BR, D), lambda i: (0, i, 0)),
            pl.BlockSpec((BR, D), lambda i: (i, 0)),
            pl.BlockSpec((BR, 1), lambda i: (i, 0)),
            pl.BlockSpec((1, D), lambda i: (0, 0)),
            pl.BlockSpec((D, D), lambda i: (0, 0)),
        ],
        out_specs=pl.BlockSpec((BR, D), lambda i: (i, 0)),
        out_shape=jax.ShapeDtypeStruct((NPAD, D), jnp.float32),
    )(acc1, y1, dinv2d, b1, W2)


def _tc_c_body(a_ref, y_ref, d_ref, b_ref, o_ref):
    o_ref[...] = (a_ref[0] + a_ref[1] + y_ref[...]) * d_ref[...] + b_ref[...]


def _tc_c(acc2, y2, dinv2d, b2):
    return pl.pallas_call(
        _tc_c_body,
        grid=(GRID_R,),
        in_specs=[
            pl.BlockSpec((NC, BR, D), lambda i: (0, i, 0)),
            pl.BlockSpec((BR, D), lambda i: (i, 0)),
            pl.BlockSpec((BR, 1), lambda i: (i, 0)),
            pl.BlockSpec((1, D), lambda i: (0, 0)),
        ],
        out_specs=pl.BlockSpec((BR, D), lambda i: (i, 0)),
        out_shape=jax.ShapeDtypeStruct((NPAD, D), jnp.float32),
    )(acc2, y2, dinv2d, b2)


def kernel(x, edge_index, edge_weight, node_type, W1, b1, W2, b2):
    del node_type
    pad = EPAD - E
    pad_dst = (N + jnp.arange(pad, dtype=jnp.int32) % (NPAD - N))
    srcp = jnp.pad(edge_index[0], (0, pad)).reshape(TOTCH, CK)
    dstp = jnp.concatenate([edge_index[1], pad_dst]).reshape(TOTCH, CK)
    ewp = jnp.pad(edge_weight, (0, pad)).reshape(TOTCH, CK)
    xp = jnp.pad(x, ((0, NPAD - N), (0, 0)))

    dinv = _sc_deg(dstp, ewp)
    dinv2d = dinv.reshape(NPAD, 1)
    y1 = _tc_a(xp, W1, dinv2d)
    acc1 = _sc_layer(y1, srcp, dstp, ewp)
    y2 = _tc_b(acc1, y1, dinv2d, b1.reshape(1, D), W2)
    acc2 = _sc_layer(y2, srcp, dstp, ewp)
    out = _tc_c(acc2, y2, dinv2d, b2.reshape(1, D))
    return out[:N]


# speedup vs baseline: 1.2051x; 1.2051x over previous
import functools

import jax
import jax.numpy as jnp
from jax import lax
from jax.experimental import pallas as pl
from jax.experimental.pallas import tpu as pltpu
from jax.experimental.pallas import tpu_sc as plsc

N = 10000
E = 320000
D = 128
NC = 2
NS = 16
NPAD = 10240
RPT = NPAD // NS
CK = 128
NCH = 79
EPT = NCH * CK
EPAD = NC * NS * EPT

_mesh = plsc.VectorSubcoreMesh(
    core_axis_name="c", subcore_axis_name="s", num_cores=NC, num_subcores=NS)


def _sc_deg_body(dst_hbm, ew_hbm, dinv_hbm, shdeg, dstv, ewv, buf):
    c = lax.axis_index("c")
    s = lax.axis_index("s")

    @pl.when(c == 0)
    def _():
        rs = s * RPT
        for k in range(RPT // 16):
            buf[pl.ds(k * 16, 16)] = jnp.zeros((16,), jnp.float32)
        pltpu.sync_copy(buf, shdeg.at[pl.ds(rs, RPT)])
        plsc.subcore_barrier()

        for c2 in range(NC):
            pltpu.sync_copy(dst_hbm.at[c2, s], dstv)
            pltpu.sync_copy(ew_hbm.at[c2, s], ewv)

            def chunk(j, carry):
                pltpu.sync_copy(ewv.at[j], shdeg.at[dstv.at[j]], add=True)
                return carry

            lax.fori_loop(0, NCH, chunk, 0)
        plsc.subcore_barrier()

        pltpu.sync_copy(shdeg.at[pl.ds(rs, RPT)], buf)
        for k in range(RPT // 16):
            v = buf[pl.ds(k * 16, 16)] + 1.0
            i = lax.bitcast_convert_type(v, jnp.int32)
            g = lax.bitcast_convert_type(
                jnp.int32(0x5F3759DF) - (i >> 1), jnp.float32)
            for _ in range(3):
                g = g * (1.5 - 0.5 * v * g * g)
            buf[pl.ds(k * 16, 16)] = g
        pltpu.sync_copy(buf, dinv_hbm.at[pl.ds(rs, RPT)])


_sc_deg = functools.partial(
    pl.kernel,
    out_type=jax.ShapeDtypeStruct((NPAD,), jnp.float32),
    mesh=_mesh,
    scratch_types=[
        pltpu.VMEM_SHARED((NPAD,), jnp.float32),
        pltpu.VMEM((NCH, CK), jnp.int32),
        pltpu.VMEM((NCH, CK), jnp.float32),
        pltpu.VMEM((RPT,), jnp.float32),
    ],
)(_sc_deg_body)


def _sc_layer_body(y_hbm, src_hbm, dst_hbm, ew_hbm, out_hbm,
                   shacc, srcv, dstv, ewv, rows, sem):
    c = lax.axis_index("c")
    s = lax.axis_index("s")
    rs = s * RPT

    @pl.when(c == 0)
    def _():
        pltpu.sync_copy(y_hbm.at[pl.ds(rs, RPT)], shacc.at[pl.ds(rs, RPT)])

    @pl.when(c == 1)
    def _():
        def zrow(r, carry):
            for k in range(D // 16):
                rows[r, pl.ds(k * 16, 16)] = jnp.zeros((16,), jnp.float32)
            return carry

        lax.fori_loop(0, CK, zrow, 0)
        for t in range(RPT // CK):
            pltpu.sync_copy(rows, shacc.at[pl.ds(rs + t * CK, CK)])

    pltpu.sync_copy(src_hbm.at[c, s], srcv)
    pltpu.sync_copy(dst_hbm.at[c, s], dstv)
    pltpu.sync_copy(ew_hbm.at[c, s], ewv)
    plsc.subcore_barrier()

    def chunk(j, carry):
        pltpu.async_copy(y_hbm.at[srcv.at[j]], rows, sem).wait()

        def edge16(eb, c2):
            wv = ewv[j, pl.ds(eb * 16, 16)]
            for l in range(16):
                w = wv[l]
                e = eb * 16 + l
                for k in range(D // 16):
                    rows[e, pl.ds(k * 16, 16)] = rows[e, pl.ds(k * 16, 16)] * w
            return c2

        lax.fori_loop(0, CK // 16, edge16, 0)
        pltpu.sync_copy(rows, shacc.at[dstv.at[j]], add=True)
        return carry

    lax.fori_loop(0, NCH, chunk, 0)
    plsc.subcore_barrier()
    pltpu.sync_copy(shacc.at[pl.ds(rs, RPT)], out_hbm.at[c, pl.ds(rs, RPT)])


_sc_layer = functools.partial(
    pl.kernel,
    out_type=jax.ShapeDtypeStruct((NC, NPAD, D), jnp.float32),
    mesh=_mesh,
    scratch_types=[
        pltpu.VMEM_SHARED((NPAD, D), jnp.float32),
        pltpu.VMEM((NCH, CK), jnp.int32),
        pltpu.VMEM((NCH, CK), jnp.int32),
        pltpu.VMEM((NCH, CK), jnp.float32),
        pltpu.VMEM((CK, D), jnp.float32),
        pltpu.SemaphoreType.DMA,
    ],
)(_sc_layer_body)


GRID_R = 8
BR = NPAD // GRID_R


def _tc_a_body(x_ref, w_ref, d_ref, y_ref):
    xw = jnp.dot(x_ref[...], w_ref[...], preferred_element_type=jnp.float32)
    y_ref[...] = xw * d_ref[...]


def _tc_a(xp, W1, dinv2d):
    return pl.pallas_call(
        _tc_a_body,
        grid=(GRID_R,),
        in_specs=[
            pl.BlockSpec((BR, D), lambda i: (i, 0)),
            pl.BlockSpec((D, D), lambda i: (0, 0)),
            pl.BlockSpec((BR, 1), lambda i: (i, 0)),
        ],
        out_specs=pl.BlockSpec((BR, D), lambda i: (i, 0)),
        out_shape=jax.ShapeDtypeStruct((NPAD, D), jnp.float32),
    )(xp, W1, dinv2d)


def _tc_b_body(a_ref, d_ref, b_ref, w_ref, y_ref):
    d = d_ref[...]
    h = jnp.maximum((a_ref[0] + a_ref[1]) * d + b_ref[...], 0.0)
    y_ref[...] = jnp.dot(
        h, w_ref[...], preferred_element_type=jnp.float32) * d


def _tc_b(acc1, dinv2d, b1, W2):
    return pl.pallas_call(
        _tc_b_body,
        grid=(GRID_R,),
        in_specs=[
            pl.BlockSpec((NC, BR, D), lambda i: (0, i, 0)),
            pl.BlockSpec((BR, 1), lambda i: (i, 0)),
            pl.BlockSpec((1, D), lambda i: (0, 0)),
            pl.BlockSpec((D, D), lambda i: (0, 0)),
        ],
        out_specs=pl.BlockSpec((BR, D), lambda i: (i, 0)),
        out_shape=jax.ShapeDtypeStruct((NPAD, D), jnp.float32),
    )(acc1, dinv2d, b1, W2)


def _tc_c_body(a_ref, d_ref, b_ref, o_ref):
    o_ref[...] = (a_ref[0] + a_ref[1]) * d_ref[...] + b_ref[...]


def _tc_c(acc2, dinv2d, b2):
    return pl.pallas_call(
        _tc_c_body,
        grid=(GRID_R,),
        in_specs=[
            pl.BlockSpec((NC, BR, D), lambda i: (0, i, 0)),
            pl.BlockSpec((BR, 1), lambda i: (i, 0)),
            pl.BlockSpec((1, D), lambda i: (0, 0)),
        ],
        out_specs=pl.BlockSpec((BR, D), lambda i: (i, 0)),
        out_shape=jax.ShapeDtypeStruct((NPAD, D), jnp.float32),
    )(acc2, dinv2d, b2)


def kernel(x, edge_index, edge_weight, node_type, W1, b1, W2, b2):
    del node_type
    pad = EPAD - E
    srcp = jnp.pad(edge_index[0], (0, pad)).reshape(NC, NS, NCH, CK)
    dstp = jnp.pad(edge_index[1], (0, pad)).reshape(NC, NS, NCH, CK)
    ewp = jnp.pad(edge_weight, (0, pad)).reshape(NC, NS, NCH, CK)
    xp = jnp.pad(x, ((0, NPAD - N), (0, 0)))

    dinv = _sc_deg(dstp, ewp)
    dinv2d = dinv.reshape(NPAD, 1)
    y1 = _tc_a(xp, W1, dinv2d)
    acc1 = _sc_layer(y1, srcp, dstp, ewp)
    y2 = _tc_b(acc1, dinv2d, b1.reshape(1, D), W2)
    acc2 = _sc_layer(y2, srcp, dstp, ewp)
    out = _tc_c(acc2, dinv2d, b2.reshape(1, D))
    return out[:N]
